# baseline (speedup 1.0000x reference)
import functools

import jax
import jax.numpy as jnp
from jax import lax
from jax.experimental import pallas as pl
from jax.experimental.pallas import tpu as pltpu
from jax.experimental.pallas import tpu_sc as plsc

B = 16384
VOCAB = 1000000
D = 64
TW = 8192
HALFROWS = 61 * TW
TBLOCKS = 62
NCTX = 5
NP = NCTX + 1
NC = 2
NS = 16
NW = NC * NS
BW = B // NW
CHUNK = 128
NCHUNK = BW // CHUNK
ROWS = CHUNK * NP
GATHER = 128
NGATHER = ROWS // GATHER
GROUPS = CHUNK // 16


_GATHER_DNUMS = lax.GatherDimensionNumbers(
    offset_dims=(), collapsed_slice_dims=(0,), start_index_map=(0,))


def _permute(x, perm):
    return lax.gather(
        x, perm.reshape(16, 1), _GATHER_DNUMS, (1,),
        mode=lax.GatherScatterMode.PROMISE_IN_BOUNDS)


def _lane_sum(m, lane):
    for shift in (8, 4, 2, 1):
        m = m + _permute(m, lane ^ shift)
    return m


def _transpose_body(xl_ref, xr_ref, out_ref):
    xl = xl_ref[...].astype(jnp.bfloat16)
    xr = xr_ref[...].astype(jnp.bfloat16)
    out_ref[...] = jnp.concatenate([xl.T, xr.T], axis=1)


_transpose_call = pl.pallas_call(
    _transpose_body,
    grid=(TBLOCKS,),
    in_specs=[
        pl.BlockSpec((D, TW), lambda i: (0, i)),
        pl.BlockSpec((D, TW), lambda i: (0, HALFROWS // TW + i)),
    ],
    out_specs=pl.BlockSpec((TW, 128), lambda i: (i, 0)),
    out_shape=jax.ShapeDtypeStruct((TBLOCKS * TW, 128), jnp.bfloat16),
)


def _score_chunk(rv, out_v, out_base):
    lane = lax.broadcasted_iota(jnp.int32, (16,), 0)
    masks = [lane == i for i in range(16)]

    def load_row_f32(row):
        halves = []
        for k in range(D // 32):
            ab = rv[row, pl.ds(k * 16, 16)]
            halves.append(plsc.bitcast(ab << 16, jnp.float32))
            halves.append(plsc.bitcast(ab & jnp.int32(-65536), jnp.float32))
        return halves

    def group_body(g, _):
        acc = [jnp.zeros((16,), jnp.float32) for _ in range(NCTX)]
        for u in range(16):
            row = (g * 16 + u) * NP
            t = load_row_f32(row)
            for j in range(NCTX):
                c = load_row_f32(row + 1 + j)
                m = c[0] * t[0]
                for k in range(1, D // 16):
                    m = m + c[k] * t[k]
                s = _lane_sum(m, lane)
                q = u * NCTX + j
                acc[q // 16] = jnp.where(masks[q % 16], s, acc[q // 16])
        for k in range(NCTX):
            out_v[pl.ds(out_base + g * (16 * NCTX) + k * 16, 16)] = acc[k]
        return 0

    lax.fori_loop(0, GROUPS, group_body, 0)


@functools.partial(
    pl.kernel,
    out_type=jax.ShapeDtypeStruct((B * NCTX,), jnp.float32),
    mesh=plsc.VectorSubcoreMesh(core_axis_name="c", subcore_axis_name="s"),
    compiler_params=pltpu.CompilerParams(use_tc_tiling_on_sc=False,
                                         needs_layout_passes=False),
    scratch_types=[
        pltpu.VMEM((BW * NP,), jnp.int32),
        pltpu.VMEM((ROWS, D // 2), jnp.int32),
        pltpu.VMEM((ROWS, D // 2), jnp.int32),
        pltpu.VMEM((BW * NCTX,), jnp.float32),
        pltpu.SemaphoreType.DMA,
        pltpu.SemaphoreType.DMA,
    ],
)
def _sc_score(idx_hbm, table_hbm, out_hbm, idx_v, rows0, rows1, out_v, sem0, sem1):
    wid = lax.axis_index("s") * NC + lax.axis_index("c")
    pltpu.sync_copy(idx_hbm.at[pl.ds(wid * (BW * NP), BW * NP)], idx_v)

    bufs = (rows0, rows1)
    sems = (sem0, sem1)

    def start_gathers(ci):
        buf = bufs[ci % 2]
        sem = sems[ci % 2]
        return [
            pltpu.async_copy(
                table_hbm.at[idx_v.at[pl.ds(ci * ROWS + gth * GATHER, GATHER)]],
                buf.at[pl.ds(gth * GATHER, GATHER)],
                sem,
            )
            for gth in range(NGATHER)
        ]

    pending = start_gathers(0)
    for ci in range(NCHUNK):
        for cp in pending:
            cp.wait()
        if ci + 1 < NCHUNK:
            pending = start_gathers(ci + 1)
        _score_chunk(bufs[ci % 2], out_v, ci * CHUNK * NCTX)

    pltpu.sync_copy(out_v, out_hbm.at[pl.ds(wid * (BW * NCTX), BW * NCTX)])


def kernel(target, context, table):
    t = target.reshape(B, 1).astype(jnp.int32)
    c = context.reshape(B, NCTX).astype(jnp.int32)
    idx6 = jnp.concatenate([t, c], axis=1).reshape(B * NP)
    idx6 = jnp.where(idx6 < HALFROWS, 2 * idx6, 2 * (idx6 - HALFROWS) + 1)
    t128 = _transpose_call(table.T, table.T)
    tbl_i32 = lax.bitcast_convert_type(
        t128.reshape(TBLOCKS * TW, 64, 2), jnp.int32)
    tbl_lin = tbl_i32.reshape(2 * TBLOCKS * TW, 32)
    out = _sc_score(idx6, tbl_lin)
    return out.reshape(B, NCTX)


# speedup vs baseline: 9.2291x; 9.2291x over previous
import functools

import jax
import jax.numpy as jnp
from jax import lax
from jax.experimental import pallas as pl
from jax.experimental.pallas import tpu as pltpu
from jax.experimental.pallas import tpu_sc as plsc

B = 16384
VOCAB = 1000000
D = 64
TW = 8192
QSTEP = 30 * TW
TBLOCKS = 33
NCTX = 5
NP = NCTX + 1
NC = 2
NS = 16
NW = NC * NS
BW = B // NW
CHUNK = 128
NCHUNK = BW // CHUNK
ROWS = CHUNK * NP
GATHER = 128
NGATHER = ROWS // GATHER
GROUPS = CHUNK // 16


_GATHER_DNUMS = lax.GatherDimensionNumbers(
    offset_dims=(), collapsed_slice_dims=(0,), start_index_map=(0,))


def _permute(x, perm):
    return lax.gather(
        x, perm.reshape(16, 1), _GATHER_DNUMS, (1,),
        mode=lax.GatherScatterMode.PROMISE_IN_BOUNDS)


def _lane_sum(m, lane):
    for shift in (8, 4, 2, 1):
        m = m + _permute(m, lane ^ shift)
    return m


def _pack_t(x):
    lo = lax.bitcast_convert_type(x[: D // 2, :], jnp.int32) + 0x8000
    hi = lax.bitcast_convert_type(x[D // 2:, :], jnp.int32) + 0x8000
    return lax.shift_right_logical(lo, 16) | (hi & jnp.int32(-65536))


def _transpose_body(x0_ref, x1_ref, x2_ref, x3_ref, out_ref):
    z = jnp.concatenate(
        [_pack_t(r[...]) for r in (x0_ref, x1_ref, x2_ref, x3_ref)], axis=0)
    out_ref[...] = z.T


_transpose_call = pl.pallas_call(
    _transpose_body,
    grid=(TBLOCKS,),
    in_specs=[
        pl.BlockSpec((D, TW), lambda i, j=j: (0, j * (QSTEP // TW) + i))
        for j in range(4)
    ],
    out_specs=pl.BlockSpec((TW, 128), lambda i: (i, 0)),
    out_shape=jax.ShapeDtypeStruct((TBLOCKS * TW, 128), jnp.int32),
)


def _score_chunk(rv, out_v, out_base):
    lane = lax.broadcasted_iota(jnp.int32, (16,), 0)
    masks = [lane == i for i in range(16)]

    def load_row_f32(row):
        halves = []
        for k in range(D // 32):
            ab = rv[row, pl.ds(k * 16, 16)]
            halves.append(plsc.bitcast(ab << 16, jnp.float32))
            halves.append(plsc.bitcast(ab & jnp.int32(-65536), jnp.float32))
        return halves

    def group_body(g, _):
        acc = [jnp.zeros((16,), jnp.float32) for _ in range(NCTX)]
        for u in range(16):
            row = (g * 16 + u) * NP
            t = load_row_f32(row)
            for j in range(NCTX):
                c = load_row_f32(row + 1 + j)
                m = c[0] * t[0]
                for k in range(1, D // 16):
                    m = m + c[k] * t[k]
                s = _lane_sum(m, lane)
                q = u * NCTX + j
                acc[q // 16] = jnp.where(masks[q % 16], s, acc[q // 16])
        for k in range(NCTX):
            out_v[pl.ds(out_base + g * (16 * NCTX) + k * 16, 16)] = acc[k]
        return 0

    lax.fori_loop(0, GROUPS, group_body, 0)


@functools.partial(
    pl.kernel,
    out_type=jax.ShapeDtypeStruct((B * NCTX,), jnp.float32),
    mesh=plsc.VectorSubcoreMesh(core_axis_name="c", subcore_axis_name="s"),
    compiler_params=pltpu.CompilerParams(use_tc_tiling_on_sc=False,
                                         needs_layout_passes=False),
    scratch_types=[
        pltpu.VMEM((BW * NP,), jnp.int32),
        pltpu.VMEM((ROWS, D // 2), jnp.int32),
        pltpu.VMEM((ROWS, D // 2), jnp.int32),
        pltpu.VMEM((BW * NCTX,), jnp.float32),
        pltpu.SemaphoreType.DMA,
        pltpu.SemaphoreType.DMA,
    ],
)
def _sc_score(idx_hbm, table_hbm, out_hbm, idx_v, rows0, rows1, out_v, sem0, sem1):
    wid = lax.axis_index("s") * NC + lax.axis_index("c")
    pltpu.sync_copy(idx_hbm.at[pl.ds(wid * (BW * NP), BW * NP)], idx_v)

    bufs = (rows0, rows1)
    sems = (sem0, sem1)

    def start_gathers(ci):
        buf = bufs[ci % 2]
        sem = sems[ci % 2]
        return [
            pltpu.async_copy(
                table_hbm.at[idx_v.at[pl.ds(ci * ROWS + gth * GATHER, GATHER)]],
                buf.at[pl.ds(gth * GATHER, GATHER)],
                sem,
            )
            for gth in range(NGATHER)
        ]

    pending = start_gathers(0)
    for ci in range(NCHUNK):
        for cp in pending:
            cp.wait()
        if ci + 1 < NCHUNK:
            pending = start_gathers(ci + 1)
        _score_chunk(bufs[ci % 2], out_v, ci * CHUNK * NCTX)

    pltpu.sync_copy(out_v, out_hbm.at[pl.ds(wid * (BW * NCTX), BW * NCTX)])


def kernel(target, context, table):
    t = target.reshape(B, 1).astype(jnp.int32)
    c = context.reshape(B, NCTX).astype(jnp.int32)
    idx6 = jnp.concatenate([t, c], axis=1).reshape(B * NP)
    q = jnp.minimum(idx6 // QSTEP, 3)
    idx6 = (idx6 - q * QSTEP) * 4 + q
    tT = table.T
    t128 = _transpose_call(tT, tT, tT, tT)
    tbl_lin = t128.reshape(4 * TBLOCKS * TW, 32)
    out = _sc_score(idx6, tbl_lin)
    return out.reshape(B, NCTX)
